# baseline (speedup 1.0000x reference)
import jax
import jax.numpy as jnp
from jax.experimental import pallas as pl

N = 10000
E = 100000
F_IN = 768
HID = 128
H1 = 4
BLK = 1000
NG = 8


def _mm_att_body(x_ref, w_ref, as_ref, ad_ref, b_ref, h_ref, als_ref, ald_ref,
                 *, heads, relu_in):
    x = x_ref[...]
    if relu_in:
        x = jnp.maximum(x + b_ref[...], 0.0)
    h = jnp.dot(x, w_ref[...], preferred_element_type=jnp.float32)
    h_ref[...] = h
    hh = h.reshape(x.shape[0], heads, HID)
    als_ref[...] = jnp.sum(hh * as_ref[...][None, :, :], axis=-1)
    ald_ref[...] = jnp.sum(hh * ad_ref[...][None, :, :], axis=-1)


def _mm_att(x, w, a_s, a_d, b_in, heads, relu_in):
    import functools
    n, k = x.shape
    m = w.shape[1]
    grid = n // BLK
    body = functools.partial(_mm_att_body, heads=heads, relu_in=relu_in)
    return pl.pallas_call(
        body,
        grid=(grid,),
        in_specs=[
            pl.BlockSpec((BLK, k), lambda i: (i, 0)),
            pl.BlockSpec((k, m), lambda i: (0, 0)),
            pl.BlockSpec((heads, HID), lambda i: (0, 0)),
            pl.BlockSpec((heads, HID), lambda i: (0, 0)),
            pl.BlockSpec((1, k), lambda i: (0, 0)),
        ],
        out_specs=[
            pl.BlockSpec((BLK, m), lambda i: (i, 0)),
            pl.BlockSpec((BLK, heads), lambda i: (i, 0)),
            pl.BlockSpec((BLK, heads), lambda i: (i, 0)),
        ],
        out_shape=[
            jax.ShapeDtypeStruct((n, m), jnp.float32),
            jax.ShapeDtypeStruct((n, heads), jnp.float32),
            jax.ShapeDtypeStruct((n, heads), jnp.float32),
        ],
    )(x, w, a_s, a_d, b_in)


def _clf_groups_body(y_ref, b_ref, wc_ref, bc_ref, batch_ref,
                     clf_ref, genum_ref, se_ref, cnt_ref):
    i = pl.program_id(0)
    h = jnp.maximum(y_ref[...] + b_ref[...], 0.0)
    z = jnp.dot(h, wc_ref[...], preferred_element_type=jnp.float32) + bc_ref[...]
    z = z - jnp.max(z, axis=1, keepdims=True)
    ez = jnp.exp(z)
    clf = ez / jnp.sum(ez, axis=1, keepdims=True)
    clf_ref[...] = clf
    bvec = batch_ref[...]
    gids = jax.lax.broadcasted_iota(jnp.int32, (BLK, NG), 1)
    m = (bvec == gids).astype(jnp.float32)

    @pl.when(i == 0)
    def _():
        genum_ref[...] = jnp.zeros_like(genum_ref)
        se_ref[...] = jnp.zeros_like(se_ref)
        cnt_ref[...] = jnp.zeros_like(cnt_ref)

    genum_ref[...] += jnp.dot(m.T, h, preferred_element_type=jnp.float32)
    se_ref[...] += jnp.dot((m * clf[:, 0:1]).T, h,
                           preferred_element_type=jnp.float32)
    cnt_ref[...] += jnp.sum(m, axis=0).reshape(NG, 1)


def _clf_groups(y, b2, wc, bc, batch2d):
    n = y.shape[0]
    grid = n // BLK
    return pl.pallas_call(
        _clf_groups_body,
        grid=(grid,),
        in_specs=[
            pl.BlockSpec((BLK, HID), lambda i: (i, 0)),
            pl.BlockSpec((1, HID), lambda i: (0, 0)),
            pl.BlockSpec((HID, 2), lambda i: (0, 0)),
            pl.BlockSpec((1, 2), lambda i: (0, 0)),
            pl.BlockSpec((BLK, 1), lambda i: (i, 0)),
        ],
        out_specs=[
            pl.BlockSpec((BLK, 2), lambda i: (i, 0)),
            pl.BlockSpec((NG, HID), lambda i: (0, 0)),
            pl.BlockSpec((NG, HID), lambda i: (0, 0)),
            pl.BlockSpec((NG, 1), lambda i: (0, 0)),
        ],
        out_shape=[
            jax.ShapeDtypeStruct((n, 2), jnp.float32),
            jax.ShapeDtypeStruct((NG, HID), jnp.float32),
            jax.ShapeDtypeStruct((NG, HID), jnp.float32),
            jax.ShapeDtypeStruct((NG, 1), jnp.float32),
        ],
    )(y, b2, wc, bc, batch2d)


def _head_body(genum_ref, cnt_ref, se_ref, agg_ref, p_ref,
               wi1a_ref, wi1b_ref, bi1_ref, wi2_ref, bi2_ref,
               wf1_ref, bf1_ref, wf2_ref, bf2_ref,
               out_ref, aggl_ref, mi_ref):
    ge = genum_ref[...] / cnt_ref[...]
    se = se_ref[...]
    gep = jnp.dot(p_ref[...], ge, preferred_element_type=jnp.float32)

    def mlp(a, b):
        z = (jnp.dot(a, wi1a_ref[...], preferred_element_type=jnp.float32)
             + jnp.dot(b, wi1b_ref[...], preferred_element_type=jnp.float32)
             + bi1_ref[...])
        z = jnp.maximum(z, 0.0)
        z = jnp.dot(z, wi2_ref[...], preferred_element_type=jnp.float32) + bi2_ref[...]
        return jnp.maximum(z, 0.0)

    jj = mlp(ge, se)
    mm = mlp(gep, se)
    mi = jnp.mean(jj) - jnp.clip(jnp.log(jnp.mean(jnp.exp(mm))),
                                 -100000.0, 100000.0)
    mi_ref[...] = mi.reshape(1, 1)

    agg = agg_ref[...]
    a00 = agg[:, 0:1]
    a01 = agg[:, 1:2]
    a10 = agg[:, 2:3]
    a11 = agg[:, 3:4]
    r0 = jnp.maximum(jnp.abs(a00) + jnp.abs(a01), 1e-05)
    r1 = jnp.maximum(jnp.abs(a10) + jnp.abs(a11), 1e-05)
    d0 = a00 / r0
    d1 = a11 / r1
    total = jnp.sum(((d0 - 1.0) ** 2 + (d1 - 1.0) ** 2) * 0.5)
    aggl_ref[...] = (total / NG).reshape(1, 1)

    o = jnp.maximum(jnp.dot(se, wf1_ref[...], preferred_element_type=jnp.float32)
                    + bf1_ref[...], 0.0)
    out_ref[...] = jnp.dot(o, wf2_ref[...], preferred_element_type=jnp.float32) + bf2_ref[...]


def _head(genum, cnt, se, agg, p, wi1a, wi1b, bi1, wi2, bi2, wf1, bf1, wf2, bf2):
    full = lambda s: pl.BlockSpec(s, lambda: tuple(0 for _ in s))
    args = [genum, cnt, se, agg, p, wi1a, wi1b, bi1, wi2, bi2, wf1, bf1, wf2, bf2]
    return pl.pallas_call(
        _head_body,
        grid=(),
        in_specs=[full(a.shape) for a in args],
        out_specs=[full((NG, 2)), full((1, 1)), full((1, 1))],
        out_shape=[
            jax.ShapeDtypeStruct((NG, 2), jnp.float32),
            jax.ShapeDtypeStruct((1, 1), jnp.float32),
            jax.ShapeDtypeStruct((1, 1), jnp.float32),
        ],
    )(*args)


def _gat_edges(h, als, ald, s, d, heads):
    n = h.shape[0]
    hh = h.reshape(n, heads, HID)
    alpha = jax.nn.leaky_relu(als[s] + ald[d], negative_slope=0.2)
    amax = jax.ops.segment_max(alpha, d, num_segments=n)
    ex = jnp.exp(alpha - amax[d])
    den = jax.ops.segment_sum(ex, d, num_segments=n)
    att = ex / (den[d] + 1e-16)
    y = jax.ops.segment_sum(hh[s] * att[:, :, None], d, num_segments=n)
    return y


def kernel(x, edge_index, batch, W1, a_src1, a_dst1, b1, W2, a_src2, a_dst2,
           b2, Wc, bc, Wf1, bf1, Wf2, bf2, Wi1, bi1, Wi2, bi2):
    src, dst = edge_index[0], edge_index[1]
    loops = jnp.arange(N, dtype=src.dtype)
    s = jnp.concatenate([src, loops])
    d = jnp.concatenate([dst, loops])

    zeros_in = jnp.zeros((1, F_IN), jnp.float32)
    h1, als1, ald1 = _mm_att(x, W1, a_src1, a_dst1, zeros_in, H1, False)
    y1 = _gat_edges(h1, als1, ald1, s, d, H1).reshape(N, H1 * HID)

    h2, als2, ald2 = _mm_att(y1, W2, a_src2, a_dst2, b1.reshape(1, -1), 1, True)
    y2 = jnp.mean(_gat_edges(h2, als2, ald2, s, d, 1), axis=1)

    clf, genum, se, cnt = _clf_groups(y2, b2.reshape(1, -1), Wc,
                                      bc.reshape(1, -1), batch.reshape(N, 1))

    gs = batch[src]
    gd = batch[dst]
    seg = jnp.where(gs == gd, gs, NG)
    vals = (clf[src][:, :, None] * clf[dst][:, None, :]).reshape(E, 4)
    agg = jax.ops.segment_sum(vals, seg, num_segments=NG + 1)[:NG]

    perm = jax.random.permutation(jax.random.key(42), NG)
    p = jax.nn.one_hot(perm, NG, dtype=jnp.float32)

    out, aggl, mi = _head(
        genum, cnt, se, agg, p,
        Wi1[:HID], Wi1[HID:], bi1.reshape(1, -1), Wi2, bi2.reshape(1, -1),
        Wf1, bf1.reshape(1, -1), Wf2, bf2.reshape(1, -1))
    return (out, aggl[0, 0], jnp.asarray(0.0, dtype=jnp.float32), mi[0, 0])


# speedup vs baseline: 2.4699x; 2.4699x over previous
import jax
import jax.numpy as jnp
from jax.experimental import pallas as pl

N = 10000
E = 100000
F_IN = 768
HID = 128
H1 = 4
BLK = 1000
NG = 8


def _mm_att_body(x_ref, w_ref, as_ref, ad_ref, b_ref, h_ref, als_ref, ald_ref,
                 *, heads, relu_in):
    x = x_ref[...]
    if relu_in:
        x = jnp.maximum(x + b_ref[...], 0.0)
    h = jnp.dot(x, w_ref[...], preferred_element_type=jnp.float32)
    h_ref[...] = h
    hh = h.reshape(x.shape[0], heads, HID)
    als_ref[...] = jnp.sum(hh * as_ref[...][None, :, :], axis=-1)
    ald_ref[...] = jnp.sum(hh * ad_ref[...][None, :, :], axis=-1)


def _mm_att(x, w, a_s, a_d, b_in, heads, relu_in):
    import functools
    n, k = x.shape
    m = w.shape[1]
    grid = n // BLK
    body = functools.partial(_mm_att_body, heads=heads, relu_in=relu_in)
    return pl.pallas_call(
        body,
        grid=(grid,),
        in_specs=[
            pl.BlockSpec((BLK, k), lambda i: (i, 0)),
            pl.BlockSpec((k, m), lambda i: (0, 0)),
            pl.BlockSpec((heads, HID), lambda i: (0, 0)),
            pl.BlockSpec((heads, HID), lambda i: (0, 0)),
            pl.BlockSpec((1, k), lambda i: (0, 0)),
        ],
        out_specs=[
            pl.BlockSpec((BLK, m), lambda i: (i, 0)),
            pl.BlockSpec((BLK, heads), lambda i: (i, 0)),
            pl.BlockSpec((BLK, heads), lambda i: (i, 0)),
        ],
        out_shape=[
            jax.ShapeDtypeStruct((n, m), jnp.float32),
            jax.ShapeDtypeStruct((n, heads), jnp.float32),
            jax.ShapeDtypeStruct((n, heads), jnp.float32),
        ],
    )(x, w, a_s, a_d, b_in)


def _clf_groups_body(y_ref, b_ref, wc_ref, bc_ref, batch_ref,
                     clf_ref, genum_ref, se_ref, cnt_ref):
    i = pl.program_id(0)
    h = jnp.maximum(y_ref[...] + b_ref[...], 0.0)
    z = jnp.dot(h, wc_ref[...], preferred_element_type=jnp.float32) + bc_ref[...]
    z = z - jnp.max(z, axis=1, keepdims=True)
    ez = jnp.exp(z)
    clf = ez / jnp.sum(ez, axis=1, keepdims=True)
    clf_ref[...] = clf
    bvec = batch_ref[...]
    gids = jax.lax.broadcasted_iota(jnp.int32, (BLK, NG), 1)
    m = (bvec == gids).astype(jnp.float32)

    @pl.when(i == 0)
    def _():
        genum_ref[...] = jnp.zeros_like(genum_ref)
        se_ref[...] = jnp.zeros_like(se_ref)
        cnt_ref[...] = jnp.zeros_like(cnt_ref)

    genum_ref[...] += jnp.dot(m.T, h, preferred_element_type=jnp.float32)
    se_ref[...] += jnp.dot((m * clf[:, 0:1]).T, h,
                           preferred_element_type=jnp.float32)
    cnt_ref[...] += jnp.sum(m, axis=0).reshape(NG, 1)


def _clf_groups(y, b2, wc, bc, batch2d):
    n = y.shape[0]
    grid = n // BLK
    return pl.pallas_call(
        _clf_groups_body,
        grid=(grid,),
        in_specs=[
            pl.BlockSpec((BLK, HID), lambda i: (i, 0)),
            pl.BlockSpec((1, HID), lambda i: (0, 0)),
            pl.BlockSpec((HID, 2), lambda i: (0, 0)),
            pl.BlockSpec((1, 2), lambda i: (0, 0)),
            pl.BlockSpec((BLK, 1), lambda i: (i, 0)),
        ],
        out_specs=[
            pl.BlockSpec((BLK, 2), lambda i: (i, 0)),
            pl.BlockSpec((NG, HID), lambda i: (0, 0)),
            pl.BlockSpec((NG, HID), lambda i: (0, 0)),
            pl.BlockSpec((NG, 1), lambda i: (0, 0)),
        ],
        out_shape=[
            jax.ShapeDtypeStruct((n, 2), jnp.float32),
            jax.ShapeDtypeStruct((NG, HID), jnp.float32),
            jax.ShapeDtypeStruct((NG, HID), jnp.float32),
            jax.ShapeDtypeStruct((NG, 1), jnp.float32),
        ],
    )(y, b2, wc, bc, batch2d)


def _head_body(genum_ref, cnt_ref, se_ref, agg_ref, p_ref,
               wi1a_ref, wi1b_ref, bi1_ref, wi2_ref, bi2_ref,
               wf1_ref, bf1_ref, wf2_ref, bf2_ref,
               out_ref, aggl_ref, mi_ref):
    ge = genum_ref[...] / cnt_ref[...]
    se = se_ref[...]
    gep = jnp.dot(p_ref[...], ge, preferred_element_type=jnp.float32)

    def mlp(a, b):
        z = (jnp.dot(a, wi1a_ref[...], preferred_element_type=jnp.float32)
             + jnp.dot(b, wi1b_ref[...], preferred_element_type=jnp.float32)
             + bi1_ref[...])
        z = jnp.maximum(z, 0.0)
        z = jnp.dot(z, wi2_ref[...], preferred_element_type=jnp.float32) + bi2_ref[...]
        return jnp.maximum(z, 0.0)

    jj = mlp(ge, se)
    mm = mlp(gep, se)
    mi = jnp.mean(jj) - jnp.clip(jnp.log(jnp.mean(jnp.exp(mm))),
                                 -100000.0, 100000.0)
    mi_ref[...] = mi.reshape(1, 1)

    agg = agg_ref[...]
    a00 = agg[:, 0:1]
    a01 = agg[:, 1:2]
    a10 = agg[:, 2:3]
    a11 = agg[:, 3:4]
    r0 = jnp.maximum(jnp.abs(a00) + jnp.abs(a01), 1e-05)
    r1 = jnp.maximum(jnp.abs(a10) + jnp.abs(a11), 1e-05)
    d0 = a00 / r0
    d1 = a11 / r1
    total = jnp.sum(((d0 - 1.0) ** 2 + (d1 - 1.0) ** 2) * 0.5)
    aggl_ref[...] = (total / NG).reshape(1, 1)

    o = jnp.maximum(jnp.dot(se, wf1_ref[...], preferred_element_type=jnp.float32)
                    + bf1_ref[...], 0.0)
    out_ref[...] = jnp.dot(o, wf2_ref[...], preferred_element_type=jnp.float32) + bf2_ref[...]


def _head(genum, cnt, se, agg, p, wi1a, wi1b, bi1, wi2, bi2, wf1, bf1, wf2, bf2):
    full = lambda s: pl.BlockSpec(s, lambda: tuple(0 for _ in s))
    args = [genum, cnt, se, agg, p, wi1a, wi1b, bi1, wi2, bi2, wf1, bf1, wf2, bf2]
    return pl.pallas_call(
        _head_body,
        grid=(),
        in_specs=[full(a.shape) for a in args],
        out_specs=[full((NG, 2)), full((1, 1)), full((1, 1))],
        out_shape=[
            jax.ShapeDtypeStruct((NG, 2), jnp.float32),
            jax.ShapeDtypeStruct((1, 1), jnp.float32),
            jax.ShapeDtypeStruct((1, 1), jnp.float32),
        ],
    )(*args)


EP = 110080
PER_W = 3440
CHUNK = 344
NP = 10112
STRIPE = 632


def _sc_scatter_add(msgs, d_pad):
    import functools
    from jax import lax
    from jax.experimental.pallas import tpu as pltpu
    from jax.experimental.pallas import tpu_sc as plsc

    heads = msgs.shape[0]
    zeros = jnp.zeros((STRIPE, HID), jnp.float32)
    mesh = plsc.VectorSubcoreMesh(core_axis_name="c", subcore_axis_name="s")

    @functools.partial(
        pl.kernel, mesh=mesh,
        out_type=jax.ShapeDtypeStruct((heads, 2, NP, HID), jnp.float32),
        scratch_types=[
            pltpu.VMEM((CHUNK,), jnp.int32),
            pltpu.VMEM((CHUNK, HID), jnp.float32),
            pltpu.VMEM_SHARED((NP, HID), jnp.float32),
        ],
    )
    def body(msgs_hbm, d_hbm, zeros_hbm, out_hbm, idx_v, rows_v, acc_sh):
        c = lax.axis_index("c")
        sub = lax.axis_index("s")
        wid = c * 16 + sub
        for h in range(heads):
            pltpu.sync_copy(zeros_hbm, acc_sh.at[pl.ds(sub * STRIPE, STRIPE)])
            plsc.subcore_barrier()
            for ci in range(PER_W // CHUNK):
                eb = wid * PER_W + ci * CHUNK
                pltpu.sync_copy(d_hbm.at[pl.ds(eb, CHUNK)], idx_v)
                pltpu.sync_copy(msgs_hbm.at[h, pl.ds(eb, CHUNK)], rows_v)
                pltpu.sync_copy(rows_v, acc_sh.at[idx_v], add=True)
            plsc.subcore_barrier()
            pltpu.sync_copy(acc_sh.at[pl.ds(sub * STRIPE, STRIPE)],
                            out_hbm.at[h, c, pl.ds(sub * STRIPE, STRIPE)])
            plsc.subcore_barrier()

    return body(msgs, d_pad, zeros)


def _gat_edges(h, als, ald, s, d, heads):
    n = h.shape[0]
    hh = h.reshape(n, heads, HID)
    alpha = jax.nn.leaky_relu(als[s] + ald[d], negative_slope=0.2)
    amax = jax.ops.segment_max(alpha, d, num_segments=n)
    ex = jnp.exp(alpha - amax[d])
    den = jax.ops.segment_sum(ex, d, num_segments=n)
    att = ex / (den[d] + 1e-16)
    msgs = hh[s] * att[:, :, None]
    msgs = jnp.transpose(msgs, (1, 0, 2))
    msgs = jnp.pad(msgs, ((0, 0), (0, EP - s.shape[0]), (0, 0)))
    d_pad = jnp.pad(d, (0, EP - d.shape[0]))
    parts = _sc_scatter_add(msgs, d_pad)
    y = (parts[:, 0] + parts[:, 1])[:, :n, :]
    return jnp.transpose(y, (1, 0, 2))


def kernel(x, edge_index, batch, W1, a_src1, a_dst1, b1, W2, a_src2, a_dst2,
           b2, Wc, bc, Wf1, bf1, Wf2, bf2, Wi1, bi1, Wi2, bi2):
    src, dst = edge_index[0], edge_index[1]
    loops = jnp.arange(N, dtype=src.dtype)
    s = jnp.concatenate([src, loops])
    d = jnp.concatenate([dst, loops])

    zeros_in = jnp.zeros((1, F_IN), jnp.float32)
    h1, als1, ald1 = _mm_att(x, W1, a_src1, a_dst1, zeros_in, H1, False)
    y1 = _gat_edges(h1, als1, ald1, s, d, H1).reshape(N, H1 * HID)

    h2, als2, ald2 = _mm_att(y1, W2, a_src2, a_dst2, b1.reshape(1, -1), 1, True)
    y2 = jnp.mean(_gat_edges(h2, als2, ald2, s, d, 1), axis=1)

    clf, genum, se, cnt = _clf_groups(y2, b2.reshape(1, -1), Wc,
                                      bc.reshape(1, -1), batch.reshape(N, 1))

    gs = batch[src]
    gd = batch[dst]
    seg = jnp.where(gs == gd, gs, NG)
    vals = (clf[src][:, :, None] * clf[dst][:, None, :]).reshape(E, 4)
    agg = jax.ops.segment_sum(vals, seg, num_segments=NG + 1)[:NG]

    perm = jax.random.permutation(jax.random.key(42), NG)
    p = jax.nn.one_hot(perm, NG, dtype=jnp.float32)

    out, aggl, mi = _head(
        genum, cnt, se, agg, p,
        Wi1[:HID], Wi1[HID:], bi1.reshape(1, -1), Wi2, bi2.reshape(1, -1),
        Wf1, bf1.reshape(1, -1), Wf2, bf2.reshape(1, -1))
    return (out, aggl[0, 0], jnp.asarray(0.0, dtype=jnp.float32), mi[0, 0])
